# baseline (speedup 1.0000x reference)
import functools

import jax
import jax.numpy as jnp
from jax import lax
from jax.experimental import pallas as pl
from jax.experimental.pallas import tpu as pltpu
from jax.experimental.pallas import tpu_sc as plsc

N = 10000
HALF = 128
NT = 16
CHUNK = 64
NCH = 160
NRND = 4
NCHR = NCH // NRND
NBUF = 4
EPT = NCH * CHUNK
EPAD = NT * EPT
ACC_ROWS = N + 16
SLAB = 632


def _slab_base(s, total):
    b = jnp.where(s < NT - 1, s * SLAB, total - SLAB)
    return pl.multiple_of(b, 8)

@functools.cache
def _mesh():
    return plsc.VectorSubcoreMesh(core_axis_name="c", subcore_axis_name="s")


def _sc_scatter_body(hpA, hpB, src4, dst4, outA, outB, sidx, didx, rows0,
                     rows1, rows2, rows3, acc, gsem0, gsem1, gsem2, gsem3,
                     ssem0, ssem1, ssem2, ssem3):
    c = lax.axis_index("c")
    s = lax.axis_index("s")
    rows = (rows0, rows1, rows2, rows3)
    gsem = (gsem0, gsem1, gsem2, gsem3)
    ssem = (ssem0, ssem1, ssem2, ssem3)

    def run(hp, out):
        base = _slab_base(s, N)
        pltpu.sync_copy(hp.at[pl.ds(base, SLAB)], acc.at[pl.ds(base, SLAB)])

        def g_start(j, b):
            pltpu.async_copy(hp.at[sidx.at[j]], rows[b], gsem[b])

        def g_wait(j, b):
            pltpu.make_async_copy(hp.at[sidx.at[j]], rows[b], gsem[b]).wait()

        def s_start(j, b):
            pltpu.async_copy(rows[b], acc.at[didx.at[j]], ssem[b], add=True)

        def s_wait(j, b):
            pltpu.make_async_copy(rows[b], acc.at[didx.at[j]], ssem[b]).wait()

        for r in range(NRND):
            pltpu.sync_copy(src4.at[s, r], sidx)
            pltpu.sync_copy(dst4.at[s, r], didx)
            if r == 0:
                plsc.subcore_barrier()

            g_start(0, 0)
            g_start(1, 1)

            @pl.loop(0, NCHR, step=NBUF)
            def _(jj):
                for b in range(NBUF):
                    j = jj + b

                    @pl.when(j > 1)
                    def _():
                        s_wait(j - 2, (b + NBUF - 2) % NBUF)

                    @pl.when(j + 2 < NCHR)
                    def _():
                        g_start(j + 2, (b + 2) % NBUF)

                    g_wait(j, b)
                    s_start(j, b)

            s_wait(NCHR - 2, (NBUF - 2) % NBUF)
            s_wait(NCHR - 1, NBUF - 1)

        plsc.subcore_barrier()
        pltpu.sync_copy(acc.at[pl.ds(base, SLAB)], out.at[pl.ds(base, SLAB)])

    @pl.when(c == 0)
    def _():
        run(hpA, outA)

    @pl.when(c == 1)
    def _():
        run(hpB, outB)


@functools.cache
def _sc_scatter():
    return pl.kernel(
        _sc_scatter_body,
        out_type=(jax.ShapeDtypeStruct((N, HALF), jnp.float32),
                  jax.ShapeDtypeStruct((N, HALF), jnp.float32)),
        mesh=_mesh(),
        scratch_types=[
            pltpu.VMEM((NCHR, CHUNK), jnp.int32),
            pltpu.VMEM((NCHR, CHUNK), jnp.int32),
            pltpu.VMEM((CHUNK, HALF), jnp.float32),
            pltpu.VMEM((CHUNK, HALF), jnp.float32),
            pltpu.VMEM((CHUNK, HALF), jnp.float32),
            pltpu.VMEM((CHUNK, HALF), jnp.float32),
            pltpu.VMEM_SHARED((ACC_ROWS, HALF), jnp.float32),
            pltpu.SemaphoreType.DMA,
            pltpu.SemaphoreType.DMA,
            pltpu.SemaphoreType.DMA,
            pltpu.SemaphoreType.DMA,
            pltpu.SemaphoreType.DMA,
            pltpu.SemaphoreType.DMA,
            pltpu.SemaphoreType.DMA,
            pltpu.SemaphoreType.DMA,
        ],
    )


def _sc_deg_body(dst4, ones_hbm, outA, outB, didx, ones_v, acc, ssem):
    c = lax.axis_index("c")
    s = lax.axis_index("s")
    base = _slab_base(s, N)

    pltpu.sync_copy(ones_hbm, acc.at[pl.ds(base, SLAB)])
    pltpu.sync_copy(ones_hbm.at[pl.ds(0, CHUNK)], ones_v)
    for q in range(NRND // 2):
        pltpu.sync_copy(dst4.at[s, c * (NRND // 2) + q], didx)
        if q == 0:
            plsc.subcore_barrier()

        @pl.loop(0, NCHR, step=8)
        def _(t0):
            for t in range(8):
                pltpu.async_copy(ones_v, acc.at[didx.at[t0 + t]], ssem,
                                 add=True)
            for t in range(8):
                pltpu.make_async_copy(ones_v, acc.at[didx.at[t0 + t]],
                                      ssem).wait()

    plsc.subcore_barrier()

    @pl.when(c == 0)
    def _():
        pltpu.sync_copy(acc.at[pl.ds(base, SLAB)], outA.at[pl.ds(base, SLAB)])

    @pl.when(c == 1)
    def _():
        pltpu.sync_copy(acc.at[pl.ds(base, SLAB)], outB.at[pl.ds(base, SLAB)])


@functools.cache
def _sc_deg():
    return pl.kernel(
        _sc_deg_body,
        out_type=(jax.ShapeDtypeStruct((N, HALF), jnp.float32),
                  jax.ShapeDtypeStruct((N, HALF), jnp.float32)),
        mesh=_mesh(),
        scratch_types=[
            pltpu.VMEM((NCHR, CHUNK), jnp.int32),
            pltpu.VMEM((CHUNK, HALF), jnp.float32),
            pltpu.VMEM_SHARED((ACC_ROWS, HALF), jnp.float32),
            pltpu.SemaphoreType.DMA,
        ],
    )


_PREC = lax.Precision.DEFAULT


def _mmu_body(x_ref, w_ref, oa_ref, ob_ref):
    h = jnp.dot(x_ref[...], w_ref[...], preferred_element_type=jnp.float32,
                precision=_PREC)
    oa_ref[...] = h[:, :HALF]
    ob_ref[...] = h[:, HALF:]


def _mmu(x, w):
    m, k = x.shape
    bm = 1000
    return pl.pallas_call(
        _mmu_body,
        grid=(m // bm,),
        in_specs=[
            pl.BlockSpec((bm, k), lambda i: (i, 0)),
            pl.BlockSpec((k, 2 * HALF), lambda i: (0, 0)),
        ],
        out_specs=[
            pl.BlockSpec((bm, HALF), lambda i: (i, 0)),
            pl.BlockSpec((bm, HALF), lambda i: (i, 0)),
        ],
        out_shape=[jax.ShapeDtypeStruct((m, HALF), jnp.float32)] * 2,
    )(x, w)


def _scale_body(a_ref, b_ref, da_ref, db_ref, oa_ref, ob_ref):
    dv = lax.rsqrt(da_ref[:, :1] + db_ref[:, :1] - 1.0)
    oa_ref[...] = a_ref[...] * dv
    ob_ref[...] = b_ref[...] * dv


def _scale(hu_a, hu_b, deg_a, deg_b):
    bm = 1000
    spec = pl.BlockSpec((bm, HALF), lambda i: (i, 0))
    return pl.pallas_call(
        _scale_body,
        grid=(N // bm,),
        in_specs=[spec] * 4,
        out_specs=[spec] * 2,
        out_shape=[jax.ShapeDtypeStruct((N, HALF), jnp.float32)] * 2,
    )(hu_a, hu_b, deg_a, deg_b)


def _bn_halves(a_ref, b_ref, dv, bias_ref, g_ref, bt_ref):
    out = []
    for half, ref in ((0, a_ref), (1, b_ref)):
        sl = pl.ds(half * HALF, HALF)
        z = ref[...] * dv + bias_ref[:, sl]
        mu = jnp.mean(z, axis=0, keepdims=True)
        zc = z - mu
        var = jnp.mean(zc * zc, axis=0, keepdims=True)
        y = zc * lax.rsqrt(var + 1e-5) * g_ref[:, sl] + bt_ref[:, sl]
        out.append(jnp.maximum(y, 0.0))
    return out


def _bn_body(a_ref, b_ref, da_ref, db_ref, bias_ref, g_ref, bt_ref, o_ref):
    dv = lax.rsqrt(da_ref[:, :1] + db_ref[:, :1] - 1.0)
    ya, yb = _bn_halves(a_ref, b_ref, dv, bias_ref, g_ref, bt_ref)
    o_ref[:, :HALF] = ya
    o_ref[:, HALF:] = yb


def _bnmm_body(a_ref, b_ref, da_ref, db_ref, bias_ref, g_ref, bt_ref, w_ref,
               oa_ref, ob_ref, s1, s2):
    p = pl.program_id(0)
    i = pl.program_id(1)
    dv = lax.rsqrt(da_ref[:, :1] + db_ref[:, :1] - 1.0)
    za = a_ref[...] * dv + bias_ref[:, :HALF]
    zb = b_ref[...] * dv + bias_ref[:, HALF:]

    @pl.when((p == 0) & (i == 0))
    def _():
        s1[...] = jnp.zeros_like(s1)
        s2[...] = jnp.zeros_like(s2)

    @pl.when(p == 0)
    def _():
        s1[:, :HALF] += jnp.sum(za, axis=0, keepdims=True)
        s1[:, HALF:] += jnp.sum(zb, axis=0, keepdims=True)
        s2[:, :HALF] += jnp.sum(za * za, axis=0, keepdims=True)
        s2[:, HALF:] += jnp.sum(zb * zb, axis=0, keepdims=True)

    @pl.when(p == 1)
    def _():
        mu = s1[...] * (1.0 / N)
        var = s2[...] * (1.0 / N) - mu * mu
        scale = lax.rsqrt(var + 1e-5) * g_ref[...]
        shift = bt_ref[...] - mu * scale
        ya = jnp.maximum(za * scale[:, :HALF] + shift[:, :HALF], 0.0)
        yb = jnp.maximum(zb * scale[:, HALF:] + shift[:, HALF:], 0.0)
        h = jnp.dot(ya * dv, w_ref[:HALF, :],
                    preferred_element_type=jnp.float32, precision=_PREC)
        h = h + jnp.dot(yb * dv, w_ref[HALF:, :],
                        preferred_element_type=jnp.float32, precision=_PREC)
        oa_ref[...] = h[:, :HALF]
        ob_ref[...] = h[:, HALF:]


def _bnmm(acc_a, acc_b, deg_a, deg_b, bias, gamma, beta, w):
    bm = 1000
    return pl.pallas_call(
        _bnmm_body,
        grid=(2, N // bm),
        in_specs=[
            pl.BlockSpec((bm, HALF), lambda p, i: (i, 0)),
            pl.BlockSpec((bm, HALF), lambda p, i: (i, 0)),
            pl.BlockSpec((bm, HALF), lambda p, i: (i, 0)),
            pl.BlockSpec((bm, HALF), lambda p, i: (i, 0)),
            pl.BlockSpec((1, 2 * HALF), lambda p, i: (0, 0)),
            pl.BlockSpec((1, 2 * HALF), lambda p, i: (0, 0)),
            pl.BlockSpec((1, 2 * HALF), lambda p, i: (0, 0)),
            pl.BlockSpec((2 * HALF, 2 * HALF), lambda p, i: (0, 0)),
        ],
        out_specs=[
            pl.BlockSpec((bm, HALF), lambda p, i: (i, 0)),
            pl.BlockSpec((bm, HALF), lambda p, i: (i, 0)),
        ],
        out_shape=[jax.ShapeDtypeStruct((N, HALF), jnp.float32)] * 2,
        scratch_shapes=[
            pltpu.VMEM((1, 2 * HALF), jnp.float32),
            pltpu.VMEM((1, 2 * HALF), jnp.float32),
        ],
    )(acc_a, acc_b, deg_a, deg_b, bias.reshape(1, -1), gamma.reshape(1, -1),
      beta.reshape(1, -1), w)


def _bn(acc_a, acc_b, deg_a, deg_b, bias, gamma, beta):
    return pl.pallas_call(
        _bn_body,
        in_specs=[
            pl.BlockSpec((N, HALF), lambda: (0, 0)),
            pl.BlockSpec((N, HALF), lambda: (0, 0)),
            pl.BlockSpec((N, HALF), lambda: (0, 0)),
            pl.BlockSpec((N, HALF), lambda: (0, 0)),
            pl.BlockSpec((1, 2 * HALF), lambda: (0, 0)),
            pl.BlockSpec((1, 2 * HALF), lambda: (0, 0)),
            pl.BlockSpec((1, 2 * HALF), lambda: (0, 0)),
        ],
        out_shape=jax.ShapeDtypeStruct((N, 2 * HALF), jnp.float32),
    )(acc_a, acc_b, deg_a, deg_b, bias.reshape(1, -1), gamma.reshape(1, -1),
      beta.reshape(1, -1))


def kernel(x, edge_index, W1, b1, gamma1, beta1, W2, b2, gamma2, beta2):
    src = edge_index[0]
    dst = edge_index[1]
    e = src.shape[0]
    npad = EPAD - e
    ar = jnp.arange(npad, dtype=jnp.int32)
    psrc = (ar * 613) % N
    pdst = N + (ar % 16)
    src4 = jnp.concatenate([src, psrc]).reshape(NT, NRND, NCHR, CHUNK)
    dst4 = jnp.concatenate([dst, pdst]).reshape(NT, NRND, NCHR, CHUNK)

    ones_slab = jnp.ones((SLAB, HALF), jnp.float32)
    deg_a, deg_b = _sc_deg()(dst4, ones_slab)

    hu_a, hu_b = _mmu(x, W1)
    hp_a, hp_b = _scale(hu_a, hu_b, deg_a, deg_b)
    acc_a, acc_b = _sc_scatter()(hp_a, hp_b, src4, dst4)
    hp_a, hp_b = _bnmm(acc_a, acc_b, deg_a, deg_b, b1, gamma1, beta1, W2)
    acc_a, acc_b = _sc_scatter()(hp_a, hp_b, src4, dst4)
    return _bn(acc_a, acc_b, deg_a, deg_b, b2, gamma2, beta2)


# speedup vs baseline: 1.0148x; 1.0148x over previous
import functools

import jax
import jax.numpy as jnp
from jax import lax
from jax.experimental import pallas as pl
from jax.experimental.pallas import tpu as pltpu
from jax.experimental.pallas import tpu_sc as plsc

N = 10000
HALF = 128
NT = 16
CHUNK = 128
NCH = 80
NRND = 2
NCHR = NCH // NRND
EPT = NCH * CHUNK
EPAD = NT * EPT
ACC_ROWS = N + 16
SLAB = 632


def _slab_base(s, total):
    b = jnp.where(s < NT - 1, s * SLAB, total - SLAB)
    return pl.multiple_of(b, 8)

@functools.cache
def _mesh():
    return plsc.VectorSubcoreMesh(core_axis_name="c", subcore_axis_name="s")


def _sc_scatter_body(hpA, hpB, src4, dst4, outA, outB, sidx, didx, rows0,
                     rows1, acc, gsem0, gsem1, ssem0, ssem1):
    c = lax.axis_index("c")
    s = lax.axis_index("s")
    rows = (rows0, rows1)
    gsem = (gsem0, gsem1)
    ssem = (ssem0, ssem1)

    def run(hp, out):
        base = _slab_base(s, N)
        pltpu.sync_copy(hp.at[pl.ds(base, SLAB)], acc.at[pl.ds(base, SLAB)])

        def g_start(j, b):
            pltpu.async_copy(hp.at[sidx.at[j]], rows[b], gsem[b])

        def g_wait(j, b):
            pltpu.make_async_copy(hp.at[sidx.at[j]], rows[b], gsem[b]).wait()

        def s_start(j, b):
            pltpu.async_copy(rows[b], acc.at[didx.at[j]], ssem[b], add=True)

        def s_wait(j, b):
            pltpu.make_async_copy(rows[b], acc.at[didx.at[j]], ssem[b]).wait()

        for r in range(NRND):
            pltpu.sync_copy(src4.at[s, r], sidx)
            pltpu.sync_copy(dst4.at[s, r], didx)
            if r == 0:
                plsc.subcore_barrier()

            g_start(0, 0)

            @pl.loop(0, NCHR, step=2)
            def _(jj):
                for b in range(2):
                    j = jj + b

                    @pl.when(j > 0)
                    def _():
                        s_wait(j - 1, 1 - b)

                    @pl.when(j + 1 < NCHR)
                    def _():
                        g_start(j + 1, 1 - b)

                    g_wait(j, b)
                    s_start(j, b)

            s_wait(NCHR - 1, 1)

        plsc.subcore_barrier()
        pltpu.sync_copy(acc.at[pl.ds(base, SLAB)], out.at[pl.ds(base, SLAB)])

    @pl.when(c == 0)
    def _():
        run(hpA, outA)

    @pl.when(c == 1)
    def _():
        run(hpB, outB)


@functools.cache
def _sc_scatter():
    return pl.kernel(
        _sc_scatter_body,
        out_type=(jax.ShapeDtypeStruct((N, HALF), jnp.float32),
                  jax.ShapeDtypeStruct((N, HALF), jnp.float32)),
        mesh=_mesh(),
        scratch_types=[
            pltpu.VMEM((NCHR, CHUNK), jnp.int32),
            pltpu.VMEM((NCHR, CHUNK), jnp.int32),
            pltpu.VMEM((CHUNK, HALF), jnp.float32),
            pltpu.VMEM((CHUNK, HALF), jnp.float32),
            pltpu.VMEM_SHARED((ACC_ROWS, HALF), jnp.float32),
            pltpu.SemaphoreType.DMA,
            pltpu.SemaphoreType.DMA,
            pltpu.SemaphoreType.DMA,
            pltpu.SemaphoreType.DMA,
        ],
    )


def _sc_deg_body(dst4, ones_hbm, outA, outB, didx, ones_v, acc, ssem):
    c = lax.axis_index("c")
    s = lax.axis_index("s")
    base = _slab_base(s, N)

    pltpu.sync_copy(ones_hbm, acc.at[pl.ds(base, SLAB)])
    pltpu.sync_copy(ones_hbm.at[pl.ds(0, CHUNK)], ones_v)
    for q in range(NRND // 2):
        pltpu.sync_copy(dst4.at[s, c * (NRND // 2) + q], didx)
        if q == 0:
            plsc.subcore_barrier()

        @pl.loop(0, NCHR, step=8)
        def _(t0):
            for t in range(8):
                pltpu.async_copy(ones_v, acc.at[didx.at[t0 + t]], ssem,
                                 add=True)
            for t in range(8):
                pltpu.make_async_copy(ones_v, acc.at[didx.at[t0 + t]],
                                      ssem).wait()

    plsc.subcore_barrier()

    @pl.when(c == 0)
    def _():
        pltpu.sync_copy(acc.at[pl.ds(base, SLAB)], outA.at[pl.ds(base, SLAB)])

    @pl.when(c == 1)
    def _():
        pltpu.sync_copy(acc.at[pl.ds(base, SLAB)], outB.at[pl.ds(base, SLAB)])


@functools.cache
def _sc_deg():
    return pl.kernel(
        _sc_deg_body,
        out_type=(jax.ShapeDtypeStruct((N, HALF), jnp.float32),
                  jax.ShapeDtypeStruct((N, HALF), jnp.float32)),
        mesh=_mesh(),
        scratch_types=[
            pltpu.VMEM((NCHR, CHUNK), jnp.int32),
            pltpu.VMEM((CHUNK, HALF), jnp.float32),
            pltpu.VMEM_SHARED((ACC_ROWS, HALF), jnp.float32),
            pltpu.SemaphoreType.DMA,
        ],
    )


_PREC = lax.Precision.DEFAULT


def _mmu_body(x_ref, w_ref, oa_ref, ob_ref):
    h = jnp.dot(x_ref[...], w_ref[...], preferred_element_type=jnp.float32,
                precision=_PREC)
    oa_ref[...] = h[:, :HALF]
    ob_ref[...] = h[:, HALF:]


def _mmu(x, w):
    m, k = x.shape
    bm = 1000
    return pl.pallas_call(
        _mmu_body,
        grid=(m // bm,),
        in_specs=[
            pl.BlockSpec((bm, k), lambda i: (i, 0)),
            pl.BlockSpec((k, 2 * HALF), lambda i: (0, 0)),
        ],
        out_specs=[
            pl.BlockSpec((bm, HALF), lambda i: (i, 0)),
            pl.BlockSpec((bm, HALF), lambda i: (i, 0)),
        ],
        out_shape=[jax.ShapeDtypeStruct((m, HALF), jnp.float32)] * 2,
    )(x, w)


def _scale_body(a_ref, b_ref, da_ref, db_ref, oa_ref, ob_ref):
    dv = lax.rsqrt(da_ref[:, :1] + db_ref[:, :1] - 1.0)
    oa_ref[...] = a_ref[...] * dv
    ob_ref[...] = b_ref[...] * dv


def _scale(hu_a, hu_b, deg_a, deg_b):
    bm = 1000
    spec = pl.BlockSpec((bm, HALF), lambda i: (i, 0))
    return pl.pallas_call(
        _scale_body,
        grid=(N // bm,),
        in_specs=[spec] * 4,
        out_specs=[spec] * 2,
        out_shape=[jax.ShapeDtypeStruct((N, HALF), jnp.float32)] * 2,
    )(hu_a, hu_b, deg_a, deg_b)


def _bn_halves(a_ref, b_ref, dv, bias_ref, g_ref, bt_ref):
    out = []
    for half, ref in ((0, a_ref), (1, b_ref)):
        sl = pl.ds(half * HALF, HALF)
        z = ref[...] * dv + bias_ref[:, sl]
        mu = jnp.mean(z, axis=0, keepdims=True)
        zc = z - mu
        var = jnp.mean(zc * zc, axis=0, keepdims=True)
        y = zc * lax.rsqrt(var + 1e-5) * g_ref[:, sl] + bt_ref[:, sl]
        out.append(jnp.maximum(y, 0.0))
    return out


def _bn_body(a_ref, b_ref, da_ref, db_ref, bias_ref, g_ref, bt_ref, o_ref):
    dv = lax.rsqrt(da_ref[:, :1] + db_ref[:, :1] - 1.0)
    ya, yb = _bn_halves(a_ref, b_ref, dv, bias_ref, g_ref, bt_ref)
    o_ref[:, :HALF] = ya
    o_ref[:, HALF:] = yb


def _bnmm_body(a_ref, b_ref, da_ref, db_ref, bias_ref, g_ref, bt_ref, w_ref,
               oa_ref, ob_ref, s1, s2):
    p = pl.program_id(0)
    i = pl.program_id(1)
    dv = lax.rsqrt(da_ref[:, :1] + db_ref[:, :1] - 1.0)
    za = a_ref[...] * dv + bias_ref[:, :HALF]
    zb = b_ref[...] * dv + bias_ref[:, HALF:]

    @pl.when((p == 0) & (i == 0))
    def _():
        s1[...] = jnp.zeros_like(s1)
        s2[...] = jnp.zeros_like(s2)

    @pl.when(p == 0)
    def _():
        s1[:, :HALF] += jnp.sum(za, axis=0, keepdims=True)
        s1[:, HALF:] += jnp.sum(zb, axis=0, keepdims=True)
        s2[:, :HALF] += jnp.sum(za * za, axis=0, keepdims=True)
        s2[:, HALF:] += jnp.sum(zb * zb, axis=0, keepdims=True)

    @pl.when(p == 1)
    def _():
        mu = s1[...] * (1.0 / N)
        var = s2[...] * (1.0 / N) - mu * mu
        scale = lax.rsqrt(var + 1e-5) * g_ref[...]
        shift = bt_ref[...] - mu * scale
        ya = jnp.maximum(za * scale[:, :HALF] + shift[:, :HALF], 0.0)
        yb = jnp.maximum(zb * scale[:, HALF:] + shift[:, HALF:], 0.0)
        h = jnp.dot(ya * dv, w_ref[:HALF, :],
                    preferred_element_type=jnp.float32, precision=_PREC)
        h = h + jnp.dot(yb * dv, w_ref[HALF:, :],
                        preferred_element_type=jnp.float32, precision=_PREC)
        oa_ref[...] = h[:, :HALF]
        ob_ref[...] = h[:, HALF:]


def _bnmm(acc_a, acc_b, deg_a, deg_b, bias, gamma, beta, w):
    bm = 1000
    return pl.pallas_call(
        _bnmm_body,
        grid=(2, N // bm),
        in_specs=[
            pl.BlockSpec((bm, HALF), lambda p, i: (i, 0)),
            pl.BlockSpec((bm, HALF), lambda p, i: (i, 0)),
            pl.BlockSpec((bm, HALF), lambda p, i: (i, 0)),
            pl.BlockSpec((bm, HALF), lambda p, i: (i, 0)),
            pl.BlockSpec((1, 2 * HALF), lambda p, i: (0, 0)),
            pl.BlockSpec((1, 2 * HALF), lambda p, i: (0, 0)),
            pl.BlockSpec((1, 2 * HALF), lambda p, i: (0, 0)),
            pl.BlockSpec((2 * HALF, 2 * HALF), lambda p, i: (0, 0)),
        ],
        out_specs=[
            pl.BlockSpec((bm, HALF), lambda p, i: (i, 0)),
            pl.BlockSpec((bm, HALF), lambda p, i: (i, 0)),
        ],
        out_shape=[jax.ShapeDtypeStruct((N, HALF), jnp.float32)] * 2,
        scratch_shapes=[
            pltpu.VMEM((1, 2 * HALF), jnp.float32),
            pltpu.VMEM((1, 2 * HALF), jnp.float32),
        ],
    )(acc_a, acc_b, deg_a, deg_b, bias.reshape(1, -1), gamma.reshape(1, -1),
      beta.reshape(1, -1), w)


def _bn(acc_a, acc_b, deg_a, deg_b, bias, gamma, beta):
    return pl.pallas_call(
        _bn_body,
        in_specs=[
            pl.BlockSpec((N, HALF), lambda: (0, 0)),
            pl.BlockSpec((N, HALF), lambda: (0, 0)),
            pl.BlockSpec((N, HALF), lambda: (0, 0)),
            pl.BlockSpec((N, HALF), lambda: (0, 0)),
            pl.BlockSpec((1, 2 * HALF), lambda: (0, 0)),
            pl.BlockSpec((1, 2 * HALF), lambda: (0, 0)),
            pl.BlockSpec((1, 2 * HALF), lambda: (0, 0)),
        ],
        out_shape=jax.ShapeDtypeStruct((N, 2 * HALF), jnp.float32),
    )(acc_a, acc_b, deg_a, deg_b, bias.reshape(1, -1), gamma.reshape(1, -1),
      beta.reshape(1, -1))


def kernel(x, edge_index, W1, b1, gamma1, beta1, W2, b2, gamma2, beta2):
    src = edge_index[0]
    dst = edge_index[1]
    e = src.shape[0]
    npad = EPAD - e
    ar = jnp.arange(npad, dtype=jnp.int32)
    psrc = (ar * 613) % N
    pdst = N + (ar % 16)
    src4 = jnp.concatenate([src, psrc]).reshape(NT, NRND, NCHR, CHUNK)
    dst4 = jnp.concatenate([dst, pdst]).reshape(NT, NRND, NCHR, CHUNK)

    ones_slab = jnp.ones((SLAB, HALF), jnp.float32)
    deg_a, deg_b = _sc_deg()(dst4, ones_slab)

    hu_a, hu_b = _mmu(x, W1)
    hp_a, hp_b = _scale(hu_a, hu_b, deg_a, deg_b)
    acc_a, acc_b = _sc_scatter()(hp_a, hp_b, src4, dst4)
    hp_a, hp_b = _bnmm(acc_a, acc_b, deg_a, deg_b, b1, gamma1, beta1, W2)
    acc_a, acc_b = _sc_scatter()(hp_a, hp_b, src4, dst4)
    return _bn(acc_a, acc_b, deg_a, deg_b, b2, gamma2, beta2)
